# Initial kernel scaffold; baseline (speedup 1.0000x reference)
import jax
import jax.numpy as jnp
from jax.experimental import pallas as pl


def kernel(story, C_0, C_1, C_2, C_3):
    raise NotImplementedError("write your pallas kernel here")


# speedup vs baseline: 9.5371x; 9.5371x over previous
import functools

import jax
import jax.numpy as jnp
from jax import lax
from jax.experimental import pallas as pl
from jax.experimental.pallas import tpu as pltpu
from jax.experimental.pallas import tpu_sc as plsc

VOCAB = 100000
DIM = 64
MAXHOPS = 3
PAD = 1
B = 1024
M = 50
T = 6
SEG = B * M
NC, NS, L = 2, 16, 16
NW = NC * NS
SEGW = SEG // NW
CH = 80
NCHUNK = SEGW // CH
ROWS = T * CH


def _sc_pooled_gather(t1, t2, t3, idx_arr):
    mesh = plsc.VectorSubcoreMesh(
        core_axis_name="c", subcore_axis_name="s",
        num_cores=NC, num_subcores=NS)
    out_t = tuple(
        jax.ShapeDtypeStruct((SEG, DIM), jnp.float32) for _ in range(3))
    scratch = [
        pltpu.VMEM((T, CH), jnp.int32),
        pltpu.VMEM((ROWS, DIM), jnp.float32),
        pltpu.VMEM((CH,), jnp.int32),
        pltpu.VMEM_SHARED((NS * CH, DIM), jnp.float32),
        pltpu.SemaphoreType.DMA,
    ]

    @functools.partial(pl.kernel, mesh=mesh, out_type=out_t,
                       scratch_types=scratch,
                       compiler_params=pltpu.CompilerParams(
                           use_tc_tiling_on_sc=False))
    def k(t1h, t2h, t3h, idx_hbm, o1, o2, o3,
          idx_v, rows_v, sidx_v, accum, sem):
        cid = lax.axis_index("c")
        sid = lax.axis_index("s")
        wid = sid * NC + cid
        for i in range(CH // L):
            sidx_v[pl.ds(i * L, L)] = (
                lax.iota(jnp.int32, L) + (sid * CH + i * L))

        for tbl, out in ((t1h, o1), (t2h, o2), (t3h, o3)):
            @pl.loop(0, NCHUNK)
            def _chunk(c, tbl=tbl, out=out):
                row = wid * NCHUNK + c
                pltpu.sync_copy(idx_hbm.at[row], idx_v)
                cps = [
                    pltpu.async_copy(tbl.at[idx_v.at[t]],
                                     rows_v.at[pl.ds(t * CH, CH)], sem)
                    for t in range(T)
                ]
                for cp in cps:
                    cp.wait()
                pltpu.sync_copy(rows_v.at[pl.ds(0, CH)],
                                accum.at[pl.ds(sid * CH, CH)])
                for t in range(1, T):
                    pltpu.sync_copy(rows_v.at[pl.ds(t * CH, CH)],
                                    accum.at[sidx_v], add=True)
                gbase = wid * SEGW + c * CH
                pltpu.sync_copy(accum.at[pl.ds(sid * CH, CH)],
                                out.at[pl.ds(gbase, CH)])

    return k(t1, t2, t3, idx_arr)


def _tc_hops(story_bmt, pads, s1, s2, s3):
    Bb = 128

    def body(st_ref, p_ref, s1_ref, s2_ref, s3_ref, o_ref):
        st = st_ref[...]
        cnt = jnp.sum((st == PAD).astype(jnp.float32), axis=2)
        p = p_ref[...]
        c3 = cnt[:, :, None]
        s1 = s1_ref[...] - c3 * p[0:1, :][None]
        s2 = s2_ref[...] - c3 * p[1:2, :][None]
        s3 = s3_ref[...] - c3 * p[2:3, :][None]
        u = jnp.mean(s1, axis=1)
        for sa, sc in ((s1, s2), (s2, s3)):
            a = jnp.sum(sa * u[:, None, :], axis=2)
            a = a - jnp.max(a, axis=1, keepdims=True)
            e = jnp.exp(a)
            pr = e / jnp.sum(e, axis=1, keepdims=True)
            u = u + jnp.sum(sc * pr[:, :, None], axis=1)
        o_ref[...] = u

    return pl.pallas_call(
        body,
        grid=(B // Bb,),
        in_specs=[
            pl.BlockSpec((Bb, M, T), lambda i: (i, 0, 0)),
            pl.BlockSpec((3, DIM), lambda i: (0, 0)),
            pl.BlockSpec((Bb, M, DIM), lambda i: (i, 0, 0)),
            pl.BlockSpec((Bb, M, DIM), lambda i: (i, 0, 0)),
            pl.BlockSpec((Bb, M, DIM), lambda i: (i, 0, 0)),
        ],
        out_specs=pl.BlockSpec((Bb, DIM), lambda i: (i, 0)),
        out_shape=jax.ShapeDtypeStruct((B, DIM), jnp.float32),
    )(story_bmt, pads, s1, s2, s3)


def kernel(story, C_0, C_1, C_2, C_3):
    story_bmt = jnp.transpose(story, (1, 0, 2))
    idx = (story_bmt.reshape(NW, NCHUNK, CH, T)
           .transpose(0, 1, 3, 2)
           .reshape(NW * NCHUNK, T, CH))
    pads = jnp.stack([C_1[PAD], C_2[PAD], C_3[PAD]], axis=0)
    S1, S2, S3 = _sc_pooled_gather(C_1, C_2, C_3, idx)
    return _tc_hops(story_bmt, pads,
                    S1.reshape(B, M, DIM),
                    S2.reshape(B, M, DIM),
                    S3.reshape(B, M, DIM))


# speedup vs baseline: 12.0675x; 1.2653x over previous
import functools

import jax
import jax.numpy as jnp
from jax import lax
from jax.experimental import pallas as pl
from jax.experimental.pallas import tpu as pltpu
from jax.experimental.pallas import tpu_sc as plsc

VOCAB = 100000
DIM = 64
PAD = 1
B = 1024
M = 50
T = 6
NC, NS, L = 2, 16, 16
NW = NC * NS
BATW = B // NW
CB = 2
CH = CB * M
NCHUNK = BATW // CB
ROWS = T * CH
JP = 32
OUTR = B * JP


def _sc_pooled_gather(t1, t2, t3, idx_arr):
    mesh = plsc.VectorSubcoreMesh(
        core_axis_name="c", subcore_axis_name="s",
        num_cores=NC, num_subcores=NS)
    out_t = tuple(
        jax.ShapeDtypeStruct((OUTR, 2 * DIM), jnp.float32) for _ in range(3))
    scratch = [
        pltpu.VMEM((NCHUNK, T, CH), jnp.int32),
        pltpu.VMEM((ROWS, DIM), jnp.float32),
        pltpu.VMEM((ROWS, DIM), jnp.float32),
        pltpu.VMEM((CH,), jnp.int32),
        pltpu.VMEM_SHARED((NS * CH, DIM), jnp.float32),
        pltpu.SemaphoreType.DMA,
        pltpu.SemaphoreType.DMA,
    ]

    @functools.partial(pl.kernel, mesh=mesh, out_type=out_t,
                       scratch_types=scratch,
                       compiler_params=pltpu.CompilerParams(
                           use_tc_tiling_on_sc=False))
    def k(t1h, t2h, t3h, idx_hbm, o1, o2, o3,
          idx_v, rows0, rows1, sidx, accum, sem0, sem1):
        cid = lax.axis_index("c")
        sid = lax.axis_index("s")
        wid = sid * NC + cid
        for i in range(CH // L):
            sidx[pl.ds(i * L, L)] = (
                lax.iota(jnp.int32, L) + (sid * CH + i * L))
        sidx[pl.ds(CH - L, L)] = (
            lax.iota(jnp.int32, L) + (sid * CH + CH - L))
        pltpu.sync_copy(idx_hbm.at[wid], idx_v)

        for tbl, out in ((t1h, o1), (t2h, o2), (t3h, o3)):
            def fire(cc, buf, sem, tbl=tbl):
                for t in range(T):
                    pltpu.async_copy(tbl.at[idx_v.at[cc, t]],
                                     buf.at[pl.ds(t * CH, CH)], sem)

            def drain(buf, sem, tbl=tbl):
                pltpu.make_async_copy(tbl.at[pl.ds(0, ROWS)], buf,
                                      sem).wait()

            def process(cc, buf, out=out):
                me = accum.at[pl.ds(sid * CH, CH)]
                pltpu.sync_copy(buf.at[pl.ds(0, CH)], me)
                for t in range(1, T):
                    pltpu.sync_copy(buf.at[pl.ds(t * CH, CH)],
                                    accum.at[sidx], add=True)
                b0 = wid * BATW + CB * cc
                for bb in range(CB):
                    for par in range(2):
                        pltpu.sync_copy(
                            accum.at[pl.ds(sid * CH + bb * M + par * 25,
                                           25)],
                            out.at[pl.ds((b0 + bb) * JP, 25),
                                   pl.ds(par * DIM, DIM)])

            fire(0, rows0, sem0)

            @pl.loop(0, NCHUNK, step=2)
            def _chunks(c):
                fire(c + 1, rows1, sem1)
                drain(rows0, sem0)
                process(c, rows0)

                @pl.when(c + 2 < NCHUNK)
                def _():
                    fire(c + 2, rows0, sem0)

                drain(rows1, sem1)
                process(c + 1, rows1)

    return k(t1, t2, t3, idx_arr)


def _tc_hops(s1, s2, s3):
    Bb = 128

    def body(s1_ref, s2_ref, s3_ref, o_ref):
        mfull = lax.broadcasted_iota(jnp.int32, (Bb, JP, 2 * DIM), 1) < 25
        m1 = lax.broadcasted_iota(jnp.int32, (Bb, JP, 1), 1) < 25
        s1 = jnp.where(mfull, s1_ref[...], 0.0)
        s2 = jnp.where(mfull, s2_ref[...], 0.0)
        s3 = jnp.where(mfull, s3_ref[...], 0.0)

        def halves(s):
            return s[:, :, 0:DIM], s[:, :, DIM:2 * DIM]

        s1e, s1o = halves(s1)
        s2e, s2o = halves(s2)
        s3e, s3o = halves(s3)
        u = (jnp.sum(s1e, axis=1, keepdims=True)
             + jnp.sum(s1o, axis=1, keepdims=True)) / float(M)
        neg = jnp.float32(-1e30)
        for ae_, ao_, ce_, co_ in ((s1e, s1o, s2e, s2o),
                                   (s2e, s2o, s3e, s3o)):
            ae = jnp.sum(ae_ * u, axis=2, keepdims=True)
            ao = jnp.sum(ao_ * u, axis=2, keepdims=True)
            ae = jnp.where(m1, ae, neg)
            ao = jnp.where(m1, ao, neg)
            mx = jnp.maximum(jnp.max(ae, axis=1, keepdims=True),
                             jnp.max(ao, axis=1, keepdims=True))
            ee = jnp.exp(ae - mx)
            eo = jnp.exp(ao - mx)
            z = jnp.sum(ee, axis=1, keepdims=True) + jnp.sum(
                eo, axis=1, keepdims=True)
            u = u + (jnp.sum(ce_ * (ee / z), axis=1, keepdims=True)
                     + jnp.sum(co_ * (eo / z), axis=1, keepdims=True))
        o_ref[...] = u

    out = pl.pallas_call(
        body,
        grid=(B // Bb,),
        in_specs=[
            pl.BlockSpec((Bb, JP, 2 * DIM), lambda i: (i, 0, 0)),
            pl.BlockSpec((Bb, JP, 2 * DIM), lambda i: (i, 0, 0)),
            pl.BlockSpec((Bb, JP, 2 * DIM), lambda i: (i, 0, 0)),
        ],
        out_specs=pl.BlockSpec((Bb, 1, DIM), lambda i: (i, 0, 0)),
        out_shape=jax.ShapeDtypeStruct((B, 1, DIM), jnp.float32),
    )(s1.reshape(B, JP, 2 * DIM),
      s2.reshape(B, JP, 2 * DIM),
      s3.reshape(B, JP, 2 * DIM))
    return out.reshape(B, DIM)


def kernel(story, C_0, C_1, C_2, C_3):
    t1 = C_1.at[PAD].set(0.0)
    t2 = C_2.at[PAD].set(0.0)
    t3 = C_3.at[PAD].set(0.0)
    story_bmt = jnp.transpose(story, (1, 0, 2))
    idx = (story_bmt.reshape(B, 25, 2, T)
           .transpose(0, 2, 1, 3)
           .reshape(NW, NCHUNK, CH, T)
           .transpose(0, 1, 3, 2))
    S1, S2, S3 = _sc_pooled_gather(t1, t2, t3, idx)
    return _tc_hops(S1, S2, S3)


# speedup vs baseline: 16.8278x; 1.3945x over previous
import functools

import jax
import jax.numpy as jnp
from jax import lax
from jax.experimental import pallas as pl
from jax.experimental.pallas import tpu as pltpu
from jax.experimental.pallas import tpu_sc as plsc

VOCAB = 100000
DIM = 64
PAD = 1
B = 1024
M = 50
T = 6
NC, NS, L = 2, 16, 16
NW = NC * NS
BATW = B // NW
NCH = M // 2
RR = 2 * T * BATW
JP = 32


def _sc_pooled_gather(t1, t2, t3, idx_arr):
    mesh = plsc.VectorSubcoreMesh(
        core_axis_name="c", subcore_axis_name="s",
        num_cores=NC, num_subcores=NS)
    out_t = tuple(
        jax.ShapeDtypeStruct((B, JP, 2 * DIM), jnp.float32)
        for _ in range(3))
    scratch = [
        pltpu.VMEM((T, M, BATW), jnp.int32),
        pltpu.VMEM((RR, DIM), jnp.float32),
        pltpu.VMEM((RR, DIM), jnp.float32),
        pltpu.VMEM((2, BATW, DIM), jnp.float32),
        pltpu.VMEM((2, BATW, DIM), jnp.float32),
        pltpu.SemaphoreType.DMA,
        pltpu.SemaphoreType.DMA,
        pltpu.SemaphoreType.DMA,
        pltpu.SemaphoreType.DMA,
    ]

    @functools.partial(pl.kernel, mesh=mesh, out_type=out_t,
                       scratch_types=scratch,
                       compiler_params=pltpu.CompilerParams(
                           use_tc_tiling_on_sc=False))
    def k(t1h, t2h, t3h, idx_hbm, o1, o2, o3,
          idx_v, rows0, rows1, accum0, accum1, semg0, semg1, semw0, semw1):
        cid = lax.axis_index("c")
        sid = lax.axis_index("s")
        wid = sid * NC + cid
        pltpu.sync_copy(
            idx_hbm.at[pl.ds(0, T), pl.ds(0, M),
                       pl.ds(wid * BATW, BATW)], idx_v)

        for tbl, out in ((t1h, o1), (t2h, o2), (t3h, o3)):
            def fire(c, buf, sem, tbl=tbl):
                for par in range(2):
                    for t in range(T):
                        pltpu.async_copy(
                            tbl.at[idx_v.at[t, 2 * c + par]],
                            buf.at[pl.ds((par * T + t) * BATW, BATW)],
                            sem)

            def drain_g(buf, sem, tbl=tbl):
                pltpu.make_async_copy(tbl.at[pl.ds(0, RR)], buf,
                                      sem).wait()

            def pool(buf, acc):
                @pl.loop(0, BATW)
                def _pool(bb):
                    for par in range(2):
                        base = par * T * BATW + bb
                        for d in range(DIM // L):
                            sl = pl.ds(d * L, L)
                            v01 = buf[base, sl] + buf[base + BATW, sl]
                            v23 = (buf[base + 2 * BATW, sl]
                                   + buf[base + 3 * BATW, sl])
                            v45 = (buf[base + 4 * BATW, sl]
                                   + buf[base + 5 * BATW, sl])
                            acc[par, bb, sl] = (v01 + v23) + v45

            def wb(c, acc, semw, out=out):
                for par in range(2):
                    pltpu.async_copy(
                        acc.at[par],
                        out.at[pl.ds(wid * BATW, BATW), c,
                               pl.ds(par * DIM, DIM)], semw)

            def drain_w(acc, semw, out=out):
                for par in range(2):
                    pltpu.make_async_copy(
                        out.at[pl.ds(0, BATW), 0, pl.ds(0, DIM)],
                        acc.at[par], semw).wait()

            fire(0, rows0, semg0)

            @pl.loop(0, NCH - 1, step=2)
            def _chunks(c):
                fire(c + 1, rows1, semg1)
                drain_g(rows0, semg0)

                @pl.when(c >= 2)
                def _():
                    drain_w(accum0, semw0)

                pool(rows0, accum0)
                wb(c, accum0, semw0)

                @pl.when(c + 2 < NCH)
                def _():
                    fire(c + 2, rows0, semg0)

                drain_g(rows1, semg1)

                @pl.when(c >= 2)
                def _():
                    drain_w(accum1, semw1)

                pool(rows1, accum1)
                wb(c + 1, accum1, semw1)

            drain_g(rows0, semg0)
            drain_w(accum0, semw0)
            pool(rows0, accum0)
            wb(NCH - 1, accum0, semw0)
            drain_w(accum0, semw0)
            drain_w(accum1, semw1)

    return k(t1, t2, t3, idx_arr)


def _tc_hops(s1, s2, s3):
    Bb = 128

    def body(s1_ref, s2_ref, s3_ref, o_ref):
        mfull = lax.broadcasted_iota(jnp.int32, (Bb, JP, 2 * DIM), 1) < 25
        m1 = lax.broadcasted_iota(jnp.int32, (Bb, JP, 1), 1) < 25
        s1 = jnp.where(mfull, s1_ref[...], 0.0)
        s2 = jnp.where(mfull, s2_ref[...], 0.0)
        s3 = jnp.where(mfull, s3_ref[...], 0.0)

        def halves(s):
            return s[:, :, 0:DIM], s[:, :, DIM:2 * DIM]

        s1e, s1o = halves(s1)
        s2e, s2o = halves(s2)
        s3e, s3o = halves(s3)
        u = (jnp.sum(s1e, axis=1, keepdims=True)
             + jnp.sum(s1o, axis=1, keepdims=True)) / float(M)
        neg = jnp.float32(-1e30)
        for ae_, ao_, ce_, co_ in ((s1e, s1o, s2e, s2o),
                                   (s2e, s2o, s3e, s3o)):
            ae = jnp.sum(ae_ * u, axis=2, keepdims=True)
            ao = jnp.sum(ao_ * u, axis=2, keepdims=True)
            ae = jnp.where(m1, ae, neg)
            ao = jnp.where(m1, ao, neg)
            mx = jnp.maximum(jnp.max(ae, axis=1, keepdims=True),
                             jnp.max(ao, axis=1, keepdims=True))
            ee = jnp.exp(ae - mx)
            eo = jnp.exp(ao - mx)
            z = jnp.sum(ee, axis=1, keepdims=True) + jnp.sum(
                eo, axis=1, keepdims=True)
            u = u + (jnp.sum(ce_ * (ee / z), axis=1, keepdims=True)
                     + jnp.sum(co_ * (eo / z), axis=1, keepdims=True))
        o_ref[...] = u

    out = pl.pallas_call(
        body,
        grid=(B // Bb,),
        in_specs=[
            pl.BlockSpec((Bb, JP, 2 * DIM), lambda i: (i, 0, 0)),
            pl.BlockSpec((Bb, JP, 2 * DIM), lambda i: (i, 0, 0)),
            pl.BlockSpec((Bb, JP, 2 * DIM), lambda i: (i, 0, 0)),
        ],
        out_specs=pl.BlockSpec((Bb, 1, DIM), lambda i: (i, 0, 0)),
        out_shape=jax.ShapeDtypeStruct((B, 1, DIM), jnp.float32),
    )(s1, s2, s3)
    return out.reshape(B, DIM)


def kernel(story, C_0, C_1, C_2, C_3):
    t1 = jnp.pad(C_1, ((0, 0), (0, DIM))).reshape(2 * VOCAB, DIM)
    t2 = jnp.pad(C_2, ((0, 0), (0, DIM))).reshape(2 * VOCAB, DIM)
    t3 = jnp.pad(C_3, ((0, 0), (0, DIM))).reshape(2 * VOCAB, DIM)
    js = jnp.transpose(story, (2, 0, 1))
    idx = jnp.where(js == PAD, 1, js * 2)
    S1, S2, S3 = _sc_pooled_gather(t1, t2, t3, idx)
    return _tc_hops(S1, S2, S3)
